# Initial kernel scaffold; baseline (speedup 1.0000x reference)
import jax
import jax.numpy as jnp
from jax.experimental import pallas as pl


def kernel(z, embeddings, is_training):
    raise NotImplementedError("write your pallas kernel here")


# speedup vs baseline: 1.5459x; 1.5459x over previous
import jax
import jax.numpy as jnp
from jax import lax
from jax.experimental import pallas as pl
from jax.experimental.pallas import tpu as pltpu

_D = 64
_K = 1024
_BLK = 512
_COMMIT = 0.25


def _vq_block(z_ref, e_ref, q_ref, idx_ref, loss_ref):
    i = pl.program_id(0)
    zb = z_ref[...]
    emb = e_ref[...]
    dots = jnp.dot(zb, emb, preferred_element_type=jnp.float32)
    rown = jnp.sum(zb * zb, axis=1, keepdims=True)
    enorm = jnp.sum(emb * emb, axis=0, keepdims=True)
    dist = (rown - 2.0 * dots) + enorm
    minv = jnp.min(dist, axis=1, keepdims=True)
    iota = lax.broadcasted_iota(jnp.int32, (_BLK, _K), 1)
    idx = jnp.min(jnp.where(dist == minv, iota, _K), axis=1)
    idx_ref[0, 0, :] = idx
    onehot = (iota == idx[:, None]).astype(jnp.float32)
    q_ref[...] = lax.dot_general(
        onehot, emb, (((1,), (1,)), ((), ())),
        preferred_element_type=jnp.float32)

    @pl.when(i == 0)
    def _init():
        loss_ref[0, 0] = 0.0

    loss_ref[0, 0] += jnp.sum(minv)

    @pl.when(i == pl.num_programs(0) - 1)
    def _fin():
        loss_ref[0, 0] = loss_ref[0, 0] * (_COMMIT / (16 * 1024 * _D))


def kernel(z, embeddings, is_training):
    zf = z.reshape(-1, _D)
    n = zf.shape[0]
    nblk = n // _BLK
    q, idx3, loss = pl.pallas_call(
        _vq_block,
        grid=(nblk,),
        in_specs=[
            pl.BlockSpec((_BLK, _D), lambda i: (i, 0)),
            pl.BlockSpec((_D, _K), lambda i: (0, 0)),
        ],
        out_specs=[
            pl.BlockSpec((_BLK, _D), lambda i: (i, 0)),
            pl.BlockSpec((1, 1, _BLK), lambda i: (i, 0, 0)),
            pl.BlockSpec(block_shape=(1, 1), index_map=lambda i: (0, 0),
                         memory_space=pltpu.SMEM),
        ],
        out_shape=[
            jax.ShapeDtypeStruct((n, _D), jnp.float32),
            jax.ShapeDtypeStruct((nblk, 1, _BLK), jnp.int32),
            jax.ShapeDtypeStruct((1, 1), jnp.float32),
        ],
    )(zf, embeddings)
    return q.reshape(z.shape), loss[0, 0], idx3.reshape(-1)
